# baseline (device time: 1098657 ns/iter reference)
import jax
import jax.numpy as jnp
from jax import lax
from jax.experimental import pallas as pl
from jax.experimental.pallas import tpu as pltpu

K = 8


def kernel(x):
    m, n = x.shape
    h = m // 2
    rp = h // K
    xb = x.astype(jnp.bfloat16)

    def body(x_hbm, out_hbm,
             y_send_buf, y_recv_buf, x_recv_buf,
             load_sems, y_send_sems, y_recv_sems,
             x_send_sems, x_recv_sems, st_y_sems, st_x_sems,
             local_sem):
        my_x = lax.axis_index("x")
        my_y = lax.axis_index("y")
        y_peer = (my_x, 1 - my_y)
        x_peer = (1 - my_x, my_y)

        barrier_sem = pltpu.get_barrier_semaphore()
        for nbr in (y_peer, x_peer):
            pl.semaphore_signal(
                barrier_sem, inc=1, device_id=nbr,
                device_id_type=pl.DeviceIdType.MESH,
            )
        pl.semaphore_wait(barrier_sem, 2)

        local = pltpu.make_async_copy(
            x_hbm, out_hbm.at[pl.ds(my_y * m, m)], local_sem
        )
        local.start()

        half_off = my_x * h
        loads = []
        for k in range(K):
            ld = pltpu.make_async_copy(
                x_hbm.at[pl.ds(half_off + k * rp, rp)],
                y_send_buf.at[k],
                load_sems.at[k],
            )
            ld.start()
            loads.append(ld)

        y_rdmas = []
        for k in range(K):
            loads[k].wait()
            r = pltpu.make_async_remote_copy(
                src_ref=y_send_buf.at[k],
                dst_ref=y_recv_buf.at[k],
                send_sem=y_send_sems.at[k],
                recv_sem=y_recv_sems.at[k],
                device_id=y_peer,
                device_id_type=pl.DeviceIdType.MESH,
            )
            r.start()
            y_rdmas.append(r)

        other_base = (1 - my_y) * m
        x_rdmas = []
        st_y = []
        for k in range(K):
            y_rdmas[k].wait_recv()
            f = pltpu.make_async_remote_copy(
                src_ref=y_recv_buf.at[k],
                dst_ref=x_recv_buf.at[k],
                send_sem=x_send_sems.at[k],
                recv_sem=x_recv_sems.at[k],
                device_id=x_peer,
                device_id_type=pl.DeviceIdType.MESH,
            )
            f.start()
            x_rdmas.append(f)
            s = pltpu.make_async_copy(
                y_recv_buf.at[k],
                out_hbm.at[pl.ds(other_base + my_x * h + k * rp, rp)],
                st_y_sems.at[k],
            )
            s.start()
            st_y.append(s)

        st_x = []
        for k in range(K):
            x_rdmas[k].wait_recv()
            s = pltpu.make_async_copy(
                x_recv_buf.at[k],
                out_hbm.at[pl.ds(other_base + (1 - my_x) * h + k * rp, rp)],
                st_x_sems.at[k],
            )
            s.start()
            st_x.append(s)

        for k in range(K):
            y_rdmas[k].wait_send()
            x_rdmas[k].wait_send()
            st_y[k].wait()
            st_x[k].wait()
        local.wait()

    return pl.pallas_call(
        body,
        out_shape=jax.ShapeDtypeStruct((2 * m, n), jnp.bfloat16),
        in_specs=[pl.BlockSpec(memory_space=pl.ANY)],
        out_specs=pl.BlockSpec(memory_space=pl.ANY),
        scratch_shapes=[
            pltpu.VMEM((K, rp, n), jnp.bfloat16),
            pltpu.VMEM((K, rp, n), jnp.bfloat16),
            pltpu.VMEM((K, rp, n), jnp.bfloat16),
            pltpu.SemaphoreType.DMA((K,)),
            pltpu.SemaphoreType.DMA((K,)),
            pltpu.SemaphoreType.DMA((K,)),
            pltpu.SemaphoreType.DMA((K,)),
            pltpu.SemaphoreType.DMA((K,)),
            pltpu.SemaphoreType.DMA((K,)),
            pltpu.SemaphoreType.DMA((K,)),
            pltpu.SemaphoreType.DMA,
        ],
        compiler_params=pltpu.CompilerParams(
            collective_id=0,
            vmem_limit_bytes=56 * 1024 * 1024,
        ),
    )(xb)


# device time: 287438 ns/iter; 3.8222x vs baseline; 3.8222x over previous
import jax
import jax.numpy as jnp
from jax import lax
from jax.experimental import pallas as pl
from jax.experimental.pallas import tpu as pltpu

K = 8


def kernel(x):
    m, n = x.shape
    h = m // 2
    rp = h // K
    xb = x.astype(jnp.bfloat16)

    def body(x_hbm, out_hbm,
             y_send_buf, y_recv_buf, x_recv_buf, own_buf,
             load_sems, y_send_sems, y_recv_sems,
             x_send_sems, x_recv_sems, st_y_sems, st_x_sems,
             st_a_sems, own_load_sems, own_store_sems):
        my_x = lax.axis_index("x")
        my_y = lax.axis_index("y")
        y_peer = (my_x, 1 - my_y)
        x_peer = (1 - my_x, my_y)

        barrier_sem = pltpu.get_barrier_semaphore()
        for nbr in (y_peer, x_peer):
            pl.semaphore_signal(
                barrier_sem, inc=1, device_id=nbr,
                device_id_type=pl.DeviceIdType.MESH,
            )
        pl.semaphore_wait(barrier_sem, 2)

        half_off = my_x * h
        own_base = my_y * m
        loads = []
        for k in range(K):
            ld = pltpu.make_async_copy(
                x_hbm.at[pl.ds(half_off + k * rp, rp)],
                y_send_buf.at[k],
                load_sems.at[k],
            )
            ld.start()
            loads.append(ld)

        own_off = (1 - my_x) * h
        NSLOT = own_buf.shape[0]
        own_loads = [None] * K
        own_stores = [None] * K
        for k in range(min(NSLOT, K)):
            own_loads[k] = pltpu.make_async_copy(
                x_hbm.at[pl.ds(own_off + k * rp, rp)],
                own_buf.at[k % NSLOT],
                own_load_sems.at[k % NSLOT],
            )
            own_loads[k].start()

        y_rdmas = []
        st_a = []
        for k in range(K):
            loads[k].wait()
            r = pltpu.make_async_remote_copy(
                src_ref=y_send_buf.at[k],
                dst_ref=y_recv_buf.at[k],
                send_sem=y_send_sems.at[k],
                recv_sem=y_recv_sems.at[k],
                device_id=y_peer,
                device_id_type=pl.DeviceIdType.MESH,
            )
            r.start()
            y_rdmas.append(r)
            s = pltpu.make_async_copy(
                y_send_buf.at[k],
                out_hbm.at[pl.ds(own_base + half_off + k * rp, rp)],
                st_a_sems.at[k],
            )
            s.start()
            st_a.append(s)

        for k in range(K):
            own_loads[k].wait()
            own_stores[k] = pltpu.make_async_copy(
                own_buf.at[k % NSLOT],
                out_hbm.at[pl.ds(own_base + own_off + k * rp, rp)],
                own_store_sems.at[k % NSLOT],
            )
            own_stores[k].start()
            nk = k + NSLOT
            if nk < K:
                own_stores[k].wait()
                own_loads[nk] = pltpu.make_async_copy(
                    x_hbm.at[pl.ds(own_off + nk * rp, rp)],
                    own_buf.at[nk % NSLOT],
                    own_load_sems.at[nk % NSLOT],
                )
                own_loads[nk].start()

        other_base = (1 - my_y) * m
        x_rdmas = []
        st_y = []
        for k in range(K):
            y_rdmas[k].wait_recv()
            f = pltpu.make_async_remote_copy(
                src_ref=y_recv_buf.at[k],
                dst_ref=x_recv_buf.at[k],
                send_sem=x_send_sems.at[k],
                recv_sem=x_recv_sems.at[k],
                device_id=x_peer,
                device_id_type=pl.DeviceIdType.MESH,
            )
            f.start()
            x_rdmas.append(f)
            s = pltpu.make_async_copy(
                y_recv_buf.at[k],
                out_hbm.at[pl.ds(other_base + my_x * h + k * rp, rp)],
                st_y_sems.at[k],
            )
            s.start()
            st_y.append(s)

        st_x = []
        for k in range(K):
            x_rdmas[k].wait_recv()
            s = pltpu.make_async_copy(
                x_recv_buf.at[k],
                out_hbm.at[pl.ds(other_base + (1 - my_x) * h + k * rp, rp)],
                st_x_sems.at[k],
            )
            s.start()
            st_x.append(s)

        for k in range(K):
            y_rdmas[k].wait_send()
            x_rdmas[k].wait_send()
            st_a[k].wait()
            st_y[k].wait()
            st_x[k].wait()
        for k in range(max(0, K - NSLOT), K):
            own_stores[k].wait()

    return pl.pallas_call(
        body,
        out_shape=jax.ShapeDtypeStruct((2 * m, n), jnp.bfloat16),
        in_specs=[pl.BlockSpec(memory_space=pl.ANY)],
        out_specs=pl.BlockSpec(memory_space=pl.ANY),
        scratch_shapes=[
            pltpu.VMEM((K, rp, n), jnp.bfloat16),
            pltpu.VMEM((K, rp, n), jnp.bfloat16),
            pltpu.VMEM((K, rp, n), jnp.bfloat16),
            pltpu.VMEM((4, rp, n), jnp.bfloat16),
            pltpu.SemaphoreType.DMA((K,)),
            pltpu.SemaphoreType.DMA((K,)),
            pltpu.SemaphoreType.DMA((K,)),
            pltpu.SemaphoreType.DMA((K,)),
            pltpu.SemaphoreType.DMA((K,)),
            pltpu.SemaphoreType.DMA((K,)),
            pltpu.SemaphoreType.DMA((K,)),
            pltpu.SemaphoreType.DMA((K,)),
            pltpu.SemaphoreType.DMA((4,)),
            pltpu.SemaphoreType.DMA((4,)),
        ],
        compiler_params=pltpu.CompilerParams(
            collective_id=0,
            vmem_limit_bytes=60 * 1024 * 1024,
        ),
    )(xb)


# device time: 259480 ns/iter; 4.2341x vs baseline; 1.1077x over previous
import jax
import jax.numpy as jnp
from jax import lax
from jax.experimental import pallas as pl
from jax.experimental.pallas import tpu as pltpu

K = 16


def kernel(x):
    m, n = x.shape
    h = m // 2
    rp = h // K

    def body(x_hbm, out_hbm,
             f32s, f32o, y_send_buf, own_bf, y_recv_buf, x_recv_buf,
             f32s_sems, f32o_sems, st_own_sems,
             y_send_sems, y_recv_sems, x_send_sems, x_recv_sems,
             st_a_sems, st_y_sems, st_x_sems):
        my_x = lax.axis_index("x")
        my_y = lax.axis_index("y")
        y_peer = (my_x, 1 - my_y)
        x_peer = (1 - my_x, my_y)

        barrier_sem = pltpu.get_barrier_semaphore()
        for nbr in (y_peer, x_peer):
            pl.semaphore_signal(
                barrier_sem, inc=1, device_id=nbr,
                device_id_type=pl.DeviceIdType.MESH,
            )
        pl.semaphore_wait(barrier_sem, 2)

        send_off = my_x * h
        keep_off = (1 - my_x) * h
        own_base = my_y * m
        other_base = (1 - my_y) * m

        def load_s(k):
            c = pltpu.make_async_copy(
                x_hbm.at[pl.ds(send_off + k * rp, rp)],
                f32s.at[k % 2], f32s_sems.at[k % 2])
            c.start()
            return c

        def load_o(k):
            c = pltpu.make_async_copy(
                x_hbm.at[pl.ds(keep_off + k * rp, rp)],
                f32o.at[k % 2], f32o_sems.at[k % 2])
            c.start()
            return c

        s_loads = [None] * K
        o_loads = [None] * K
        for k in range(2):
            s_loads[k] = load_s(k)
            o_loads[k] = load_o(k)

        y_rdmas = [None] * K
        st_a = [None] * K
        own_stores = [None] * K
        for k in range(K):
            s_loads[k].wait()
            y_send_buf[k, :, :] = f32s[k % 2, :, :].astype(jnp.bfloat16)
            if k + 2 < K:
                s_loads[k + 2] = load_s(k + 2)
            r = pltpu.make_async_remote_copy(
                src_ref=y_send_buf.at[k],
                dst_ref=y_recv_buf.at[k],
                send_sem=y_send_sems.at[k],
                recv_sem=y_recv_sems.at[k],
                device_id=y_peer,
                device_id_type=pl.DeviceIdType.MESH,
            )
            r.start()
            y_rdmas[k] = r
            sa = pltpu.make_async_copy(
                y_send_buf.at[k],
                out_hbm.at[pl.ds(own_base + send_off + k * rp, rp)],
                st_a_sems.at[k])
            sa.start()
            st_a[k] = sa

            o_loads[k].wait()
            if k >= 2:
                own_stores[k - 2].wait()
            own_bf[k % 2, :, :] = f32o[k % 2, :, :].astype(jnp.bfloat16)
            if k + 2 < K:
                o_loads[k + 2] = load_o(k + 2)
            so = pltpu.make_async_copy(
                own_bf.at[k % 2],
                out_hbm.at[pl.ds(own_base + keep_off + k * rp, rp)],
                st_own_sems.at[k % 2])
            so.start()
            own_stores[k] = so

        x_rdmas = [None] * K
        st_y = [None] * K
        for k in range(K):
            y_rdmas[k].wait_recv()
            f = pltpu.make_async_remote_copy(
                src_ref=y_recv_buf.at[k],
                dst_ref=x_recv_buf.at[k],
                send_sem=x_send_sems.at[k],
                recv_sem=x_recv_sems.at[k],
                device_id=x_peer,
                device_id_type=pl.DeviceIdType.MESH,
            )
            f.start()
            x_rdmas[k] = f
            s = pltpu.make_async_copy(
                y_recv_buf.at[k],
                out_hbm.at[pl.ds(other_base + send_off + k * rp, rp)],
                st_y_sems.at[k])
            s.start()
            st_y[k] = s

        st_x = [None] * K
        for k in range(K):
            x_rdmas[k].wait_recv()
            s = pltpu.make_async_copy(
                x_recv_buf.at[k],
                out_hbm.at[pl.ds(other_base + keep_off + k * rp, rp)],
                st_x_sems.at[k])
            s.start()
            st_x[k] = s

        for k in range(K):
            y_rdmas[k].wait_send()
            x_rdmas[k].wait_send()
            st_a[k].wait()
            st_y[k].wait()
            st_x[k].wait()
        own_stores[K - 2].wait()
        own_stores[K - 1].wait()

    return pl.pallas_call(
        body,
        out_shape=jax.ShapeDtypeStruct((2 * m, n), jnp.bfloat16),
        in_specs=[pl.BlockSpec(memory_space=pl.ANY)],
        out_specs=pl.BlockSpec(memory_space=pl.ANY),
        scratch_shapes=[
            pltpu.VMEM((2, rp, n), jnp.float32),
            pltpu.VMEM((2, rp, n), jnp.float32),
            pltpu.VMEM((K, rp, n), jnp.bfloat16),
            pltpu.VMEM((2, rp, n), jnp.bfloat16),
            pltpu.VMEM((K, rp, n), jnp.bfloat16),
            pltpu.VMEM((K, rp, n), jnp.bfloat16),
            pltpu.SemaphoreType.DMA((2,)),
            pltpu.SemaphoreType.DMA((2,)),
            pltpu.SemaphoreType.DMA((2,)),
            pltpu.SemaphoreType.DMA((K,)),
            pltpu.SemaphoreType.DMA((K,)),
            pltpu.SemaphoreType.DMA((K,)),
            pltpu.SemaphoreType.DMA((K,)),
            pltpu.SemaphoreType.DMA((K,)),
            pltpu.SemaphoreType.DMA((K,)),
            pltpu.SemaphoreType.DMA((K,)),
        ],
        compiler_params=pltpu.CompilerParams(
            collective_id=0,
            vmem_limit_bytes=62 * 1024 * 1024,
        ),
    )(x)
